# Initial kernel scaffold; baseline (speedup 1.0000x reference)
import jax
import jax.numpy as jnp
from jax.experimental import pallas as pl


def kernel(p32, p16, p8, p4, W_hid, b_hid, W_cls, b_cls, W_reg, b_reg):
    raise NotImplementedError("write your pallas kernel here")


# speedup vs baseline: 1.5727x; 1.5727x over previous
import functools

import jax
import jax.numpy as jnp
from jax import lax
from jax.experimental import pallas as pl
from jax.experimental.pallas import tpu as pltpu
from jax.experimental.pallas import tpu_sc as plsc

_C = 256
_HWS = ((16, 16), (32, 32), (64, 64))
_STRIDES = (32.0, 16.0, 8.0)
_ANCHORS = (32.0, 64.0, 128.0)
_N = 16128
_KEEP = 64
_R = 14
_NW = 32


def _rpn_body(H, W, stride, feat_ref, wh_ref, bh_ref, wcr_ref, bcr_ref,
              obj_ref, x1_ref, y1_ref, x2_ref, y2_ref, ch_ref, ar_ref):
    HW = H * W
    X = feat_ref[0]
    h0 = lax.dot_general(wh_ref[...], X, (((0,), (0,)), ((), ())),
                         preferred_element_type=jnp.float32) + bh_ref[...]
    h = jnp.where(h0 >= 0, h0, 0.01 * h0)
    cr = lax.dot_general(wcr_ref[...], h, (((0,), (0,)), ((), ())),
                         preferred_element_type=jnp.float32) + bcr_ref[...]
    cls = cr[0:3, :]
    reg = cr[3:15, :]
    obj_ref[0] = jax.nn.sigmoid(cls)
    iota = lax.broadcasted_iota(jnp.int32, (1, HW), 1)
    wmod = lax.rem(iota, W)
    cx = wmod.astype(jnp.float32) + 0.5
    cy = (iota - wmod).astype(jnp.float32) * (1.0 / W) + 0.5
    x1s, y1s, x2s, y2s, chs = [], [], [], [], []
    for a in range(3):
        bwa = _ANCHORS[a] / stride
        d0 = reg[4 * a + 0:4 * a + 1, :]
        d1 = reg[4 * a + 1:4 * a + 2, :]
        d2 = reg[4 * a + 2:4 * a + 3, :]
        d3 = reg[4 * a + 3:4 * a + 4, :]
        ncx = cx + d0 * bwa
        ncy = cy + d1 * bwa
        nw = bwa * jnp.exp(jnp.clip(d2, -4.0, 4.0))
        nh = bwa * jnp.exp(jnp.clip(d3, -4.0, 4.0))
        scx = ncx * stride
        scy = ncy * stride
        sw = nw * stride
        sh = nh * stride
        x1s.append(scx - sw / 2)
        y1s.append(scy - sh / 2)
        x2s.append(scx + sw / 2)
        y2s.append(scy + sh / 2)
        chs.append(jnp.clip(jnp.floor(3.0 + jnp.log2(jnp.sqrt(sw * sh) / 224.0)),
                            1.0, 4.0))
    x1 = jnp.concatenate(x1s, axis=0)
    y1 = jnp.concatenate(y1s, axis=0)
    x2 = jnp.concatenate(x2s, axis=0)
    y2 = jnp.concatenate(y2s, axis=0)
    x1_ref[0] = x1
    y1_ref[0] = y1
    x2_ref[0] = x2
    y2_ref[0] = y2
    ch_ref[0] = jnp.concatenate(chs, axis=0)
    ar_ref[0] = jnp.maximum(x2 - x1, 0.0) * jnp.maximum(y2 - y1, 0.0)


def _nms_body(sc_ref, x1_ref, y1_ref, x2_ref, y2_ref, ar_ref, ch_ref,
              i00_ref, i01_ref, i10_ref, i11_ref,
              w00_ref, w01_ref, w10_ref, w11_ref):
    nl_f = (pl.program_id(1) + 1).astype(jnp.float32)
    s0 = sc_ref[0]
    x1 = x1_ref[0]
    y1 = y1_ref[0]
    x2 = x2_ref[0]
    y2 = y2_ref[0]
    ar = ar_ref[0]
    ch = ch_ref[0]
    s_init = jnp.where(ch == nl_f, s0, -1.0)
    r_i = lax.broadcasted_iota(jnp.int32, (126, 128), 0)
    c_i = lax.broadcasted_iota(jnp.int32, (126, 128), 1)
    flat = r_i * 128 + c_i
    kiota = lax.broadcasted_iota(jnp.int32, (_KEEP, 1), 0)

    def body(i, carry):
        s, kx1, ky1, kx2, ky2 = carry
        m = jnp.max(s)
        j = jnp.min(jnp.where(s == m, flat, _N))
        oh = flat == j
        big = jnp.float32(-3.0e38)
        x1j = jnp.max(jnp.where(oh, x1, big))
        y1j = jnp.max(jnp.where(oh, y1, big))
        x2j = jnp.max(jnp.where(oh, x2, big))
        y2j = jnp.max(jnp.where(oh, y2, big))
        arj = jnp.max(jnp.where(oh, ar, big))
        upd = kiota == i
        kx1 = jnp.where(upd, x1j, kx1)
        ky1 = jnp.where(upd, y1j, ky1)
        kx2 = jnp.where(upd, x2j, kx2)
        ky2 = jnp.where(upd, y2j, ky2)
        xx1 = jnp.maximum(x1j, x1)
        yy1 = jnp.maximum(y1j, y1)
        xx2 = jnp.minimum(x2j, x2)
        yy2 = jnp.minimum(y2j, y2)
        inter = jnp.maximum(xx2 - xx1, 0.0) * jnp.maximum(yy2 - yy1, 0.0)
        iou = inter / (arj + ar - inter + 1e-6)
        s = jnp.where(iou > 0.5, -1e9, s)
        return (s, kx1, ky1, kx2, ky2)

    z = jnp.zeros((_KEEP, 1), jnp.float32)
    _, kx1, ky1, kx2, ky2 = lax.fori_loop(0, _KEEP, body, (s_init, z, z, z, z))

    pid1 = pl.program_id(1)
    Wi = jnp.where(pid1 == 0, 16, jnp.where(pid1 == 1, 32, 64))
    b3 = lambda v: lax.broadcast_in_dim(v, (_KEEP, _R, _R), (0, 1))
    ii_q = lax.broadcasted_iota(jnp.int32, (_KEEP, _R, _R), 2).astype(jnp.float32)
    ii_p = lax.broadcasted_iota(jnp.int32, (_KEEP, _R, _R), 1).astype(jnp.float32)
    bw = jnp.maximum(kx2 - kx1, 1.0)
    bh = jnp.maximum(ky2 - ky1, 1.0)
    gx = b3(kx1) + (ii_q + 0.5) * b3(bw) / jnp.float32(_R)
    gy = b3(ky1) + (ii_p + 0.5) * b3(bh) / jnp.float32(_R)
    x0f = jnp.floor(gx)
    wx = gx - x0f
    y0f = jnp.floor(gy)
    wy = gy - y0f
    x0i = jnp.clip(x0f.astype(jnp.int32), 0, Wi - 1)
    x1i = jnp.minimum(x0i + 1, Wi - 1)
    y0i = jnp.clip(y0f.astype(jnp.int32), 0, Wi - 1)
    y1i = jnp.minimum(y0i + 1, Wi - 1)
    i00_ref[0, 0] = y0i * Wi + x0i
    i01_ref[0, 0] = y0i * Wi + x1i
    i10_ref[0, 0] = y1i * Wi + x0i
    i11_ref[0, 0] = y1i * Wi + x1i
    w00_ref[0, 0] = (1.0 - wy) * (1.0 - wx)
    w01_ref[0, 0] = (1.0 - wy) * wx
    w10_ref[0, 0] = wy * (1.0 - wx)
    w11_ref[0, 0] = wy * wx


def _sc_roialign(tables, idxs, wgts, rows):
    per_w = rows // _NW
    chunk = 56
    nchunk = per_w // chunk
    mesh = plsc.VectorSubcoreMesh(core_axis_name="c", subcore_axis_name="s")
    out_types = [jax.ShapeDtypeStruct((rows, _C), jnp.float32)] * 3
    nrows = rows
    scratch = [
        pltpu.VMEM((4 * chunk,), jnp.int32),
        pltpu.VMEM((4 * chunk * 16,), jnp.float32),
        pltpu.VMEM((4, chunk, _C), jnp.float32),
        pltpu.VMEM((chunk, _C), jnp.float32),
        pltpu.SemaphoreType.DMA,
    ]

    @functools.partial(pl.kernel, mesh=mesh, out_type=out_types,
                       scratch_types=scratch)
    def k(t1, t2, t3, i1, i2, i3, g1, g2, g3, o1, o2, o3,
          idx_v, w_v, taps_v, out_v, sem):
        wid = lax.axis_index("s") * 2 + lax.axis_index("c")
        base0 = wid * per_w
        for tbl, ih, wh, oh in ((t1, i1, g1, o1), (t2, i2, g2, o2),
                                (t3, i3, g3, o3)):
            def chunk_body(ci, _, tbl=tbl, ih=ih, wh=wh, oh=oh):
                base = base0 + ci * chunk
                for t in range(4):
                    pltpu.sync_copy(ih.at[pl.ds(t * nrows + base, chunk)],
                                    idx_v.at[pl.ds(t * chunk, chunk)])
                    pltpu.sync_copy(wh.at[pl.ds((t * nrows + base) * 16, chunk * 16)],
                                    w_v.at[pl.ds(t * chunk * 16, chunk * 16)])
                for t in range(4):
                    pltpu.async_copy(tbl.at[idx_v.at[pl.ds(t * chunk, chunk)]],
                                     taps_v.at[t], sem).wait()

                def row_body(r, _):
                    w0 = w_v[pl.ds((0 * chunk + r) * 16, 16)]
                    w1 = w_v[pl.ds((1 * chunk + r) * 16, 16)]
                    w2 = w_v[pl.ds((2 * chunk + r) * 16, 16)]
                    w3 = w_v[pl.ds((3 * chunk + r) * 16, 16)]
                    for cc in range(_C // 16):
                        sl = pl.ds(cc * 16, 16)
                        acc = w0 * taps_v[0, r, sl]
                        acc = acc + w1 * taps_v[1, r, sl]
                        acc = acc + w2 * taps_v[2, r, sl]
                        acc = acc + w3 * taps_v[3, r, sl]
                        out_v[r, sl] = acc
                    return 0

                lax.fori_loop(0, chunk, row_body, 0)
                pltpu.sync_copy(out_v, oh.at[pl.ds(base, chunk)])
                return 0

            lax.fori_loop(0, nchunk, chunk_body, 0)

    return k(tables[0], tables[1], tables[2], idxs[0], idxs[1], idxs[2],
             wgts[0], wgts[1], wgts[2])


def kernel(p32, p16, p8, p4, W_hid, b_hid, W_cls, b_cls, W_reg, b_reg):
    B = p32.shape[0]
    feats = (p32, p16, p8)
    per = []
    for l in range(3):
        H, W = _HWS[l]
        HW = H * W
        f2 = feats[l].reshape(B, _C, HW)
        wcr = jnp.concatenate([W_cls[l], W_reg[l]], axis=1)
        bcr = jnp.concatenate([b_cls[l], b_reg[l]], axis=0).reshape(15, 1)
        outs = pl.pallas_call(
            functools.partial(_rpn_body, H, W, _STRIDES[l]),
            grid=(B,),
            in_specs=[
                pl.BlockSpec((1, _C, HW), lambda b: (b, 0, 0)),
                pl.BlockSpec((_C, _C), lambda b: (0, 0)),
                pl.BlockSpec((_C, 1), lambda b: (0, 0)),
                pl.BlockSpec((_C, 15), lambda b: (0, 0)),
                pl.BlockSpec((15, 1), lambda b: (0, 0)),
            ],
            out_specs=[pl.BlockSpec((1, 3, HW), lambda b: (b, 0, 0))] * 7,
            out_shape=[jax.ShapeDtypeStruct((B, 3, HW), jnp.float32)] * 7,
        )(f2, W_hid[l], b_hid[l].reshape(_C, 1), wcr, bcr)
        per.append([o.transpose(0, 2, 1).reshape(B, HW * 3) for o in outs])
    cat = [jnp.concatenate([per[l][k] for l in range(3)], axis=1)
           .reshape(B, 126, 128) for k in range(7)]
    obj, x1, y1, x2, y2, ch, ar = cat
    tap_shape = (B, 3, _KEEP, _R, _R)
    tap_out = pl.pallas_call(
        _nms_body,
        grid=(B, 3),
        in_specs=[pl.BlockSpec((1, 126, 128), lambda b, n: (b, 0, 0))] * 7,
        out_specs=[pl.BlockSpec((1, 1, _KEEP, _R, _R),
                                lambda b, n: (b, n, 0, 0, 0))] * 8,
        out_shape=([jax.ShapeDtypeStruct(tap_shape, jnp.int32)] * 4
                   + [jax.ShapeDtypeStruct(tap_shape, jnp.float32)] * 4),
    )(obj, x1, y1, x2, y2, ar, ch)
    idx4 = jnp.stack(tap_out[:4])
    wgt4 = jnp.stack(tap_out[4:])
    rows = B * _KEEP * _R * _R
    tables, idxs, wgts = [], [], []
    for l in range(3):
        H, W = _HWS[l]
        HW = H * W
        tables.append(feats[l].transpose(0, 2, 3, 1).reshape(B * HW, _C))
        off = (jnp.arange(B, dtype=jnp.int32) * HW)[None, :, None, None, None]
        idxs.append((idx4[:, :, l] + off).reshape(4 * rows))
        wgts.append(jnp.broadcast_to(wgt4[:, :, l].reshape(4 * rows, 1),
                                     (4 * rows, 16)).reshape(-1))
    lvl_outs = _sc_roialign(tables, idxs, wgts, rows)
    outs = []
    for o in lvl_outs:
        outs.append(o.reshape(B, _KEEP, _R * _R, _C)
                    .transpose(0, 1, 3, 2).reshape(B, _KEEP, _C, _R, _R))
    return jnp.concatenate(outs, axis=1)


# speedup vs baseline: 1.6405x; 1.0432x over previous
import functools

import jax
import jax.numpy as jnp
from jax import lax
from jax.experimental import pallas as pl
from jax.experimental.pallas import tpu as pltpu
from jax.experimental.pallas import tpu_sc as plsc

_C = 256
_HWS = ((16, 16), (32, 32), (64, 64))
_STRIDES = (32.0, 16.0, 8.0)
_ANCHORS = (32.0, 64.0, 128.0)
_N = 16128
_KEEP = 64
_R = 14
_NW = 32


def _rpn_body(H, W, stride, feat_ref, wh_ref, bh_ref, wcr_ref, bcr_ref,
              obj_ref, x1_ref, y1_ref, x2_ref, y2_ref, ch_ref, ar_ref):
    HW = H * W
    X = feat_ref[0]
    h0 = lax.dot_general(wh_ref[...], X, (((0,), (0,)), ((), ())),
                         preferred_element_type=jnp.float32) + bh_ref[...]
    h = jnp.where(h0 >= 0, h0, 0.01 * h0)
    cr = lax.dot_general(wcr_ref[...], h, (((0,), (0,)), ((), ())),
                         preferred_element_type=jnp.float32) + bcr_ref[...]
    cls = cr[0:3, :]
    reg = cr[3:15, :]
    obj_ref[0] = jax.nn.sigmoid(cls)
    iota = lax.broadcasted_iota(jnp.int32, (1, HW), 1)
    wmod = lax.rem(iota, W)
    cx = wmod.astype(jnp.float32) + 0.5
    cy = (iota - wmod).astype(jnp.float32) * (1.0 / W) + 0.5
    x1s, y1s, x2s, y2s, chs = [], [], [], [], []
    for a in range(3):
        bwa = _ANCHORS[a] / stride
        d0 = reg[4 * a + 0:4 * a + 1, :]
        d1 = reg[4 * a + 1:4 * a + 2, :]
        d2 = reg[4 * a + 2:4 * a + 3, :]
        d3 = reg[4 * a + 3:4 * a + 4, :]
        ncx = cx + d0 * bwa
        ncy = cy + d1 * bwa
        nw = bwa * jnp.exp(jnp.clip(d2, -4.0, 4.0))
        nh = bwa * jnp.exp(jnp.clip(d3, -4.0, 4.0))
        scx = ncx * stride
        scy = ncy * stride
        sw = nw * stride
        sh = nh * stride
        x1s.append(scx - sw / 2)
        y1s.append(scy - sh / 2)
        x2s.append(scx + sw / 2)
        y2s.append(scy + sh / 2)
        chs.append(jnp.clip(jnp.floor(3.0 + jnp.log2(jnp.sqrt(sw * sh) / 224.0)),
                            1.0, 4.0))
    x1 = jnp.concatenate(x1s, axis=0)
    y1 = jnp.concatenate(y1s, axis=0)
    x2 = jnp.concatenate(x2s, axis=0)
    y2 = jnp.concatenate(y2s, axis=0)
    x1_ref[0] = x1
    y1_ref[0] = y1
    x2_ref[0] = x2
    y2_ref[0] = y2
    ch_ref[0] = jnp.concatenate(chs, axis=0)
    ar_ref[0] = jnp.maximum(x2 - x1, 0.0) * jnp.maximum(y2 - y1, 0.0)


def _nms_body(sc_ref, x1_ref, y1_ref, x2_ref, y2_ref, ar_ref, ch_ref,
              i00_ref, i01_ref, i10_ref, i11_ref,
              w00_ref, w01_ref, w10_ref, w11_ref):
    nl_f = (pl.program_id(1) + 1).astype(jnp.float32)
    s0 = sc_ref[0]
    x1 = x1_ref[0]
    y1 = y1_ref[0]
    x2 = x2_ref[0]
    y2 = y2_ref[0]
    ar = ar_ref[0]
    ch = ch_ref[0]
    s_init = jnp.where(ch == nl_f, s0, -1.0)
    r_i = lax.broadcasted_iota(jnp.int32, (126, 128), 0)
    c_i = lax.broadcasted_iota(jnp.int32, (126, 128), 1)
    flat = r_i * 128 + c_i
    kiota = lax.broadcasted_iota(jnp.int32, (_KEEP, 1), 0)

    def body(i, carry):
        s, kx1, ky1, kx2, ky2 = carry
        m = jnp.max(s)
        j = jnp.min(jnp.where(s == m, flat, _N))
        oh = flat == j
        big = jnp.float32(-3.0e38)
        x1j = jnp.max(jnp.where(oh, x1, big))
        y1j = jnp.max(jnp.where(oh, y1, big))
        x2j = jnp.max(jnp.where(oh, x2, big))
        y2j = jnp.max(jnp.where(oh, y2, big))
        arj = jnp.max(jnp.where(oh, ar, big))
        upd = kiota == i
        kx1 = jnp.where(upd, x1j, kx1)
        ky1 = jnp.where(upd, y1j, ky1)
        kx2 = jnp.where(upd, x2j, kx2)
        ky2 = jnp.where(upd, y2j, ky2)
        xx1 = jnp.maximum(x1j, x1)
        yy1 = jnp.maximum(y1j, y1)
        xx2 = jnp.minimum(x2j, x2)
        yy2 = jnp.minimum(y2j, y2)
        inter = jnp.maximum(xx2 - xx1, 0.0) * jnp.maximum(yy2 - yy1, 0.0)
        iou = inter / (arj + ar - inter + 1e-6)
        s = jnp.where(iou > 0.5, -1e9, s)
        return (s, kx1, ky1, kx2, ky2)

    z = jnp.zeros((_KEEP, 1), jnp.float32)
    _, kx1, ky1, kx2, ky2 = lax.fori_loop(0, _KEEP, body, (s_init, z, z, z, z))

    pid1 = pl.program_id(1)
    Wi = jnp.where(pid1 == 0, 16, jnp.where(pid1 == 1, 32, 64))
    b3 = lambda v: lax.broadcast_in_dim(v, (_KEEP, _R, _R), (0, 1))
    ii_q = lax.broadcasted_iota(jnp.int32, (_KEEP, _R, _R), 2).astype(jnp.float32)
    ii_p = lax.broadcasted_iota(jnp.int32, (_KEEP, _R, _R), 1).astype(jnp.float32)
    bw = jnp.maximum(kx2 - kx1, 1.0)
    bh = jnp.maximum(ky2 - ky1, 1.0)
    gx = b3(kx1) + (ii_q + 0.5) * b3(bw) / jnp.float32(_R)
    gy = b3(ky1) + (ii_p + 0.5) * b3(bh) / jnp.float32(_R)
    x0f = jnp.floor(gx)
    wx = gx - x0f
    y0f = jnp.floor(gy)
    wy = gy - y0f
    x0i = jnp.clip(x0f.astype(jnp.int32), 0, Wi - 1)
    x1i = jnp.minimum(x0i + 1, Wi - 1)
    y0i = jnp.clip(y0f.astype(jnp.int32), 0, Wi - 1)
    y1i = jnp.minimum(y0i + 1, Wi - 1)
    i00_ref[0, 0] = y0i * Wi + x0i
    i01_ref[0, 0] = y0i * Wi + x1i
    i10_ref[0, 0] = y1i * Wi + x0i
    i11_ref[0, 0] = y1i * Wi + x1i
    w00_ref[0, 0] = (1.0 - wy) * (1.0 - wx)
    w01_ref[0, 0] = (1.0 - wy) * wx
    w10_ref[0, 0] = wy * (1.0 - wx)
    w11_ref[0, 0] = wy * wx


def _sc_roialign(tables, idxs, wgts, rows):
    per_w = rows // _NW
    chunk = 16
    nchunk = per_w // chunk
    mesh = plsc.VectorSubcoreMesh(core_axis_name="c", subcore_axis_name="s")
    out_types = [jax.ShapeDtypeStruct((rows, _C), jnp.float32)] * 3
    scratch = [
        pltpu.VMEM((4 * per_w,), jnp.int32),
        pltpu.VMEM((4 * per_w * 16,), jnp.float32),
        pltpu.VMEM((2, 4, chunk, _C), jnp.float32),
        pltpu.VMEM((2, chunk, _C), jnp.float32),
        pltpu.SemaphoreType.DMA,
        pltpu.SemaphoreType.DMA,
        pltpu.SemaphoreType.DMA,
        pltpu.SemaphoreType.DMA,
    ]

    @functools.partial(pl.kernel, mesh=mesh, out_type=out_types,
                       scratch_types=scratch)
    def k(t1, t2, t3, i1, i2, i3, g1, g2, g3, o1, o2, o3,
          idx_v, w_v, taps_v, out_v, sg0, sg1, so0, so1):
        wid = lax.axis_index("s") * 2 + lax.axis_index("c")
        sgs = (sg0, sg1)
        sos = (so0, so1)
        for tbl, ih, wh, oh in ((t1, i1, g1, o1), (t2, i2, g2, o2),
                                (t3, i3, g3, o3)):
            pltpu.sync_copy(ih.at[pl.ds(wid * (4 * per_w), 4 * per_w)], idx_v)
            pltpu.sync_copy(wh.at[pl.ds(wid * (4 * per_w * 16), 4 * per_w * 16)],
                            w_v)

            def fire(ci, buf, tbl=tbl):
                for t in range(4):
                    pltpu.async_copy(
                        tbl.at[idx_v.at[pl.ds(t * per_w + ci * chunk, chunk)]],
                        taps_v.at[buf, t], sgs[buf])

            fire(0, 0)

            def step(ci, b, tbl=tbl, oh=oh):
                @pl.when(ci + 1 < nchunk)
                def _():
                    fire(ci + 1, 1 - b)

                for t in range(4):
                    pltpu.make_async_copy(
                        tbl.at[idx_v.at[pl.ds(t * per_w, chunk)]],
                        taps_v.at[b, t], sgs[b]).wait()
                @pl.when(ci >= 2)
                def _():
                    pltpu.make_async_copy(
                        out_v.at[b], oh.at[pl.ds(wid * per_w, chunk)],
                        sos[b]).wait()

                def row_body(r, _, b=b, ci=ci):
                    w0 = w_v[pl.ds((0 * per_w + ci * chunk + r) * 16, 16)]
                    w1 = w_v[pl.ds((1 * per_w + ci * chunk + r) * 16, 16)]
                    w2 = w_v[pl.ds((2 * per_w + ci * chunk + r) * 16, 16)]
                    w3 = w_v[pl.ds((3 * per_w + ci * chunk + r) * 16, 16)]
                    for cc in range(_C // 16):
                        sl = pl.ds(cc * 16, 16)
                        acc = w0 * taps_v[b, 0, r, sl]
                        acc = acc + w1 * taps_v[b, 1, r, sl]
                        acc = acc + w2 * taps_v[b, 2, r, sl]
                        acc = acc + w3 * taps_v[b, 3, r, sl]
                        out_v[b, r, sl] = acc
                    return 0

                lax.fori_loop(0, chunk, row_body, 0)
                pltpu.async_copy(
                    out_v.at[b],
                    oh.at[pl.ds(wid * per_w + ci * chunk, chunk)], sos[b])

            def pair_body(c2, _):
                step(c2 * 2, 0)
                step(c2 * 2 + 1, 1)
                return 0

            lax.fori_loop(0, nchunk // 2, pair_body, 0)
            step(jnp.int32(nchunk - 1), 0)
            for b in range(2):
                pltpu.make_async_copy(
                    out_v.at[b], oh.at[pl.ds(wid * per_w, chunk)],
                    sos[b]).wait()

    return k(tables[0], tables[1], tables[2], idxs[0], idxs[1], idxs[2],
             wgts[0], wgts[1], wgts[2])


def kernel(p32, p16, p8, p4, W_hid, b_hid, W_cls, b_cls, W_reg, b_reg):
    B = p32.shape[0]
    feats = (p32, p16, p8)
    per = []
    for l in range(3):
        H, W = _HWS[l]
        HW = H * W
        f2 = feats[l].reshape(B, _C, HW)
        wcr = jnp.concatenate([W_cls[l], W_reg[l]], axis=1)
        bcr = jnp.concatenate([b_cls[l], b_reg[l]], axis=0).reshape(15, 1)
        outs = pl.pallas_call(
            functools.partial(_rpn_body, H, W, _STRIDES[l]),
            grid=(B,),
            in_specs=[
                pl.BlockSpec((1, _C, HW), lambda b: (b, 0, 0)),
                pl.BlockSpec((_C, _C), lambda b: (0, 0)),
                pl.BlockSpec((_C, 1), lambda b: (0, 0)),
                pl.BlockSpec((_C, 15), lambda b: (0, 0)),
                pl.BlockSpec((15, 1), lambda b: (0, 0)),
            ],
            out_specs=[pl.BlockSpec((1, 3, HW), lambda b: (b, 0, 0))] * 7,
            out_shape=[jax.ShapeDtypeStruct((B, 3, HW), jnp.float32)] * 7,
        )(f2, W_hid[l], b_hid[l].reshape(_C, 1), wcr, bcr)
        per.append([o.transpose(0, 2, 1).reshape(B, HW * 3) for o in outs])
    cat = [jnp.concatenate([per[l][k] for l in range(3)], axis=1)
           .reshape(B, 126, 128) for k in range(7)]
    obj, x1, y1, x2, y2, ch, ar = cat
    tap_shape = (B, 3, _KEEP, _R, _R)
    tap_out = pl.pallas_call(
        _nms_body,
        grid=(B, 3),
        in_specs=[pl.BlockSpec((1, 126, 128), lambda b, n: (b, 0, 0))] * 7,
        out_specs=[pl.BlockSpec((1, 1, _KEEP, _R, _R),
                                lambda b, n: (b, n, 0, 0, 0))] * 8,
        out_shape=([jax.ShapeDtypeStruct(tap_shape, jnp.int32)] * 4
                   + [jax.ShapeDtypeStruct(tap_shape, jnp.float32)] * 4),
    )(obj, x1, y1, x2, y2, ar, ch)
    idx4 = jnp.stack(tap_out[:4])
    wgt4 = jnp.stack(tap_out[4:])
    rows = B * _KEEP * _R * _R
    tables, idxs, wgts = [], [], []
    for l in range(3):
        H, W = _HWS[l]
        HW = H * W
        tables.append(feats[l].transpose(0, 2, 3, 1).reshape(B * HW, _C))
        off = (jnp.arange(B, dtype=jnp.int32) * HW)[None, :, None, None, None]
        per_w = rows // _NW
        il = (idx4[:, :, l] + off).reshape(4, _NW, per_w)
        idxs.append(il.transpose(1, 0, 2).reshape(-1))
        wl = wgt4[:, :, l].reshape(4, _NW, per_w).transpose(1, 0, 2)
        wgts.append(jnp.broadcast_to(wl[..., None],
                                     (_NW, 4, per_w, 16)).reshape(-1))
    lvl_outs = _sc_roialign(tables, idxs, wgts, rows)
    outs = []
    for o in lvl_outs:
        outs.append(o.reshape(B, _KEEP, _R * _R, _C)
                    .transpose(0, 1, 3, 2).reshape(B, _KEEP, _C, _R, _R))
    return jnp.concatenate(outs, axis=1)
